# Initial kernel scaffold; baseline (speedup 1.0000x reference)
import jax
import jax.numpy as jnp
from jax.experimental import pallas as pl


def kernel(x1, edge_index1, occ_batch1, x2, edge_index2, occ_batch2, Wself, Wneigh, b, Wp1, bp1, Wp2, bp2):
    raise NotImplementedError("write your pallas kernel here")


# speedup vs baseline: 2.8861x; 2.8861x over previous
import functools

import jax
import jax.numpy as jnp
from jax import lax
from jax.experimental import pallas as pl
from jax.experimental.pallas import tpu as pltpu
from jax.experimental.pallas import tpu_sc as plsc

N = 10000
E = 320000
D = 128
PROJ = 64
NSONG = 64
TEMP = 0.5

NC = 2
NS = 16
NW = NC * NS
EPW = E // NW
CH = 80
NCH = EPW // CH
BR = 624
TAIL = N - NS * BR

RB = 1000


def _sc_mesh():
    return plsc.VectorSubcoreMesh(core_axis_name="c", subcore_axis_name="s")


def _fill_idx(idxbuf, start, n):
    for k in range(n // 16):
        idxbuf[pl.ds(16 * k, 16)] = (start + 16 * k) + lax.iota(jnp.int32, 16)


_PIECES = [(k * CH, CH) for k in range(BR // CH)] + [(BR - BR % CH, BR % CH)]


def _init_shared(sid, rows, idx80, idx64, idx16, sh):
    row0 = sid * BR
    for (o, n) in _PIECES:
        buf = {CH: idx80, 64: idx64, TAIL: idx16}[n]
        _fill_idx(buf, row0 + o, n)
        pltpu.sync_copy(rows.at[pl.ds(0, n)], sh.at[buf])

    @pl.when(sid == NS - 1)
    def _():
        _fill_idx(idx16, NS * BR, TAIL)
        pltpu.sync_copy(rows.at[pl.ds(0, TAIL)], sh.at[idx16])


def _write_shared(cid, sid, rows, idx80, idx64, idx16, sh, out):
    row0 = pl.multiple_of(sid * BR, 8)
    for (o, n) in _PIECES:
        buf = {CH: idx80, 64: idx64, TAIL: idx16}[n]
        _fill_idx(buf, row0 + o, n)
        pltpu.sync_copy(sh.at[buf], rows.at[pl.ds(0, n)])
        pltpu.sync_copy(rows.at[pl.ds(0, n)], out.at[cid].at[pl.ds(row0 + o, n)])

    @pl.when(sid == NS - 1)
    def _():
        _fill_idx(idx16, NS * BR, TAIL)
        pltpu.sync_copy(sh.at[idx16], rows.at[pl.ds(0, TAIL)])
        pltpu.sync_copy(rows.at[pl.ds(0, TAIL)],
                        out.at[cid].at[pl.ds(NS * BR, TAIL)])


def _spmm_kernel():
    def body(h_hbm, src_hbm, dst_hbm, zrows_hbm, agg_out,
             srcb, dstb, idx64, idx16, rows, sem, agg_sh):
        cid = lax.axis_index("c")
        sid = lax.axis_index("s")
        pltpu.sync_copy(zrows_hbm, rows)
        _init_shared(sid, rows, srcb, idx64, idx16, agg_sh)
        plsc.subcore_barrier()

        base = (sid * NC + cid) * EPW

        def chunk(i, carry):
            off = pl.multiple_of(base + i * CH, 8)
            pltpu.sync_copy(src_hbm.at[pl.ds(off, CH)], srcb)
            pltpu.sync_copy(dst_hbm.at[pl.ds(off, CH)], dstb)
            pltpu.async_copy(h_hbm.at[srcb], rows, sem).wait()
            pltpu.sync_copy(rows, agg_sh.at[dstb], add=True)
            return carry

        lax.fori_loop(0, NCH, chunk, 0)
        plsc.subcore_barrier()
        _write_shared(cid, sid, rows, srcb, idx64, idx16, agg_sh, agg_out)

    return pl.kernel(
        body,
        out_type=(jax.ShapeDtypeStruct((NC, N, D), jnp.float32),),
        mesh=_sc_mesh(),
        scratch_types=(
            pltpu.VMEM((CH,), jnp.int32),
            pltpu.VMEM((CH,), jnp.int32),
            pltpu.VMEM((64,), jnp.int32),
            pltpu.VMEM((TAIL,), jnp.int32),
            pltpu.VMEM((CH, D), jnp.float32),
            pltpu.SemaphoreType.DMA,
            pltpu.VMEM_SHARED((N, D), jnp.float32),
        ))


def _deg_kernel():
    def body(dst_hbm, zrows_hbm, ones_hbm, deg_out,
             dstb, idx64, idx16, rows, onesb, deg_sh):
        cid = lax.axis_index("c")
        sid = lax.axis_index("s")
        pltpu.sync_copy(zrows_hbm, rows)
        pltpu.sync_copy(ones_hbm, onesb)
        _init_shared(sid, rows, dstb, idx64, idx16, deg_sh)
        plsc.subcore_barrier()

        base = (sid * NC + cid) * EPW

        def chunk(i, carry):
            off = pl.multiple_of(base + i * CH, 8)
            pltpu.sync_copy(dst_hbm.at[pl.ds(off, CH)], dstb)
            pltpu.sync_copy(onesb, deg_sh.at[dstb], add=True)
            return carry

        lax.fori_loop(0, NCH, chunk, 0)
        plsc.subcore_barrier()
        _write_shared(cid, sid, rows, dstb, idx64, idx16, deg_sh, deg_out)

    return pl.kernel(
        body,
        out_type=(jax.ShapeDtypeStruct((NC, N, D), jnp.float32),),
        mesh=_sc_mesh(),
        scratch_types=(
            pltpu.VMEM((CH,), jnp.int32),
            pltpu.VMEM((64,), jnp.int32),
            pltpu.VMEM((TAIL,), jnp.int32),
            pltpu.VMEM((CH, D), jnp.float32),
            pltpu.VMEM((CH, D), jnp.float32),
            pltpu.VMEM_SHARED((N, D), jnp.float32),
        ))


@functools.cache
def _get_spmm():
    return _spmm_kernel()


@functools.cache
def _get_deg():
    return _deg_kernel()


def _spmm_call(h, src, dst, zrows):
    return _get_spmm()(h, src, dst, zrows)[0]


def _deg_call(dst, zrows, ones):
    return _get_deg()(dst, zrows, ones)[0]


def _dense_body(h_ref, agg_ref, deg_ref, ws_ref, wn_ref, b_ref, out_ref, *, relu):
    agg = agg_ref[0] + agg_ref[1]
    deg = jnp.maximum(deg_ref[0, :, 0:1] + deg_ref[1, :, 0:1], 1.0)
    y = (jnp.dot(h_ref[...], ws_ref[...], preferred_element_type=jnp.float32)
         + jnp.dot(agg / deg, wn_ref[...], preferred_element_type=jnp.float32)
         + b_ref[...])
    out_ref[...] = jnp.maximum(y, 0.0) if relu else y


def _dense(h, aggp, degp, ws, wn, bias, relu):
    return pl.pallas_call(
        functools.partial(_dense_body, relu=relu),
        grid=(N // RB,),
        in_specs=[
            pl.BlockSpec((RB, D), lambda i: (i, 0)),
            pl.BlockSpec((NC, RB, D), lambda i: (0, i, 0)),
            pl.BlockSpec((NC, RB, D), lambda i: (0, i, 0)),
            pl.BlockSpec((D, D), lambda i: (0, 0)),
            pl.BlockSpec((D, D), lambda i: (0, 0)),
            pl.BlockSpec((1, D), lambda i: (0, 0)),
        ],
        out_specs=pl.BlockSpec((RB, D), lambda i: (i, 0)),
        out_shape=jax.ShapeDtypeStruct((N, D), jnp.float32),
    )(h, aggp, degp, ws, wn, bias)


def _final_body(h_ref, agg_ref, deg_ref, occ_ref, ws_ref, wn_ref, b_ref,
                s_ref, c_ref):
    i = pl.program_id(0)

    @pl.when(i == 0)
    def _():
        s_ref[...] = jnp.zeros_like(s_ref)
        c_ref[...] = jnp.zeros_like(c_ref)

    agg = agg_ref[0] + agg_ref[1]
    deg = jnp.maximum(deg_ref[0, :, 0:1] + deg_ref[1, :, 0:1], 1.0)
    y = (jnp.dot(h_ref[...], ws_ref[...], preferred_element_type=jnp.float32)
         + jnp.dot(agg / deg, wn_ref[...], preferred_element_type=jnp.float32)
         + b_ref[...])
    occ = occ_ref[0, 0, :]
    onehot = (occ[None, :] ==
              lax.broadcasted_iota(jnp.int32, (NSONG, RB), 0)).astype(jnp.float32)
    s_ref[...] += jnp.dot(onehot, y, preferred_element_type=jnp.float32)
    c_ref[...] = c_ref[...] + jnp.sum(onehot, axis=1, keepdims=True)


def _final(h, aggp, degp, occ3, ws, wn, bias):
    return pl.pallas_call(
        _final_body,
        grid=(N // RB,),
        in_specs=[
            pl.BlockSpec((RB, D), lambda i: (i, 0)),
            pl.BlockSpec((NC, RB, D), lambda i: (0, i, 0)),
            pl.BlockSpec((NC, RB, D), lambda i: (0, i, 0)),
            pl.BlockSpec((1, 1, RB), lambda i: (i, 0, 0)),
            pl.BlockSpec((D, D), lambda i: (0, 0)),
            pl.BlockSpec((D, D), lambda i: (0, 0)),
            pl.BlockSpec((1, D), lambda i: (0, 0)),
        ],
        out_specs=[pl.BlockSpec((NSONG, D), lambda i: (0, 0)),
                   pl.BlockSpec((NSONG, D), lambda i: (0, 0))],
        out_shape=[jax.ShapeDtypeStruct((NSONG, D), jnp.float32),
                   jax.ShapeDtypeStruct((NSONG, D), jnp.float32)],
    )(h, aggp, degp, occ3, ws, wn, bias)


def _head_body(s1_ref, c1_ref, s2_ref, c2_ref,
               wp1_ref, bp1_ref, wp2_ref, bp2_ref, out_ref):
    s1 = s1_ref[...] / jnp.maximum(c1_ref[...], 1.0)
    s2 = s2_ref[...] / jnp.maximum(c2_ref[...], 1.0)

    def proj(s):
        t = jnp.maximum(
            jnp.dot(s, wp1_ref[...], preferred_element_type=jnp.float32)
            + bp1_ref[...], 0.0)
        return (jnp.dot(t, wp2_ref[...], preferred_element_type=jnp.float32)
                + bp2_ref[...])

    z = jnp.concatenate([proj(s1), proj(s2)], axis=0)
    z = z / jnp.sqrt(jnp.sum(z * z, axis=1, keepdims=True))
    sim = lax.dot_general(z, z, (((1,), (1,)), ((), ())),
                          preferred_element_type=jnp.float32) / TEMP
    n2 = 2 * NSONG
    row = lax.broadcasted_iota(jnp.int32, (n2, n2), 0)
    col = lax.broadcasted_iota(jnp.int32, (n2, n2), 1)
    sim = jnp.where(row == col, -1000000000.0, sim)
    m = jnp.max(sim, axis=1, keepdims=True)
    logp = sim - (jnp.log(jnp.sum(jnp.exp(sim - m), axis=1, keepdims=True)) + m)
    lbl = jnp.where(row < NSONG, row + NSONG, row - NSONG)
    loss = -jnp.sum(jnp.where(col == lbl, logp, 0.0)) / n2
    out_ref[...] = jnp.reshape(loss, (1, 1))


def _head(s1, c1, s2, c2, wp1, bp1, wp2, bp2):
    full = lambda s: pl.BlockSpec(s, lambda: (0,) * len(s))
    return pl.pallas_call(
        _head_body,
        in_specs=[full((NSONG, D)), full((NSONG, D)),
                  full((NSONG, D)), full((NSONG, D)),
                  full((D, D)), full((1, D)), full((D, PROJ)), full((1, PROJ))],
        out_specs=full((1, 1)),
        out_shape=jax.ShapeDtypeStruct((1, 1), jnp.float32),
    )(s1, c1, s2, c2, wp1, bp1, wp2, bp2)


def kernel(x1, edge_index1, occ_batch1, x2, edge_index2, occ_batch2,
           Wself, Wneigh, b, Wp1, bp1, Wp2, bp2):
    zrows = jnp.zeros((CH, D), jnp.float32)
    ones = jnp.ones((CH, D), jnp.float32)

    def song_emb(x, ei, occ):
        src = ei[0].astype(jnp.int32)
        dst = ei[1].astype(jnp.int32)
        degp = _deg_call(dst, zrows, ones)
        aggp = _spmm_call(x, src, dst, zrows)
        h = _dense(x, aggp, degp, Wself[0], Wneigh[0], b[0][None], True)
        aggp2 = _spmm_call(h, src, dst, zrows)
        h = _dense(h, aggp2, degp, Wself[1], Wneigh[1], b[1][None], True)
        aggp3 = _spmm_call(h, src, dst, zrows)
        occ3 = occ.astype(jnp.int32).reshape(N // RB, 1, RB)
        return _final(h, aggp3, degp, occ3, Wself[2], Wneigh[2], b[2][None])

    s1, c1 = song_emb(x1, edge_index1, occ_batch1)
    s2, c2 = song_emb(x2, edge_index2, occ_batch2)
    loss = _head(s1, c1, s2, c2, Wp1, bp1[None], Wp2, bp2[None])
    return loss[0, 0]


# speedup vs baseline: 5.4086x; 1.8740x over previous
import functools

import jax
import jax.numpy as jnp
from jax import lax
from jax.experimental import pallas as pl
from jax.experimental.pallas import tpu as pltpu
from jax.experimental.pallas import tpu_sc as plsc

N = 10000
E = 320000
D = 128
PROJ = 64
NSONG = 64
TEMP = 0.5

NC = 2
NS = 16
NW = NC * NS
EPW = E // NW
CH = 80
NCH = EPW // CH
BR = 624
TAIL = N - NS * BR

RB = 1000


def _sc_mesh():
    return plsc.VectorSubcoreMesh(core_axis_name="c", subcore_axis_name="s")


def _fill_idx(idxbuf, start, n):
    for k in range(n // 16):
        idxbuf[pl.ds(16 * k, 16)] = (start + 16 * k) + lax.iota(jnp.int32, 16)


_PIECES = [(k * CH, CH) for k in range(BR // CH)] + [(BR - BR % CH, BR % CH)]


def _init_shared(sid, rows, idx80, idx64, idx16, sh):
    row0 = sid * BR
    for (o, n) in _PIECES:
        buf = {CH: idx80, 64: idx64, TAIL: idx16}[n]
        _fill_idx(buf, row0 + o, n)
        pltpu.sync_copy(rows.at[pl.ds(0, n)], sh.at[buf])

    @pl.when(sid == NS - 1)
    def _():
        _fill_idx(idx16, NS * BR, TAIL)
        pltpu.sync_copy(rows.at[pl.ds(0, TAIL)], sh.at[idx16])


def _write_shared(cid, sid, rows, idx80, idx64, idx16, sh, out):
    row0 = pl.multiple_of(sid * BR, 8)
    for (o, n) in _PIECES:
        buf = {CH: idx80, 64: idx64, TAIL: idx16}[n]
        _fill_idx(buf, row0 + o, n)
        pltpu.sync_copy(sh.at[buf], rows.at[pl.ds(0, n)])
        pltpu.sync_copy(rows.at[pl.ds(0, n)], out.at[cid].at[pl.ds(row0 + o, n)])

    @pl.when(sid == NS - 1)
    def _():
        _fill_idx(idx16, NS * BR, TAIL)
        pltpu.sync_copy(sh.at[idx16], rows.at[pl.ds(0, TAIL)])
        pltpu.sync_copy(rows.at[pl.ds(0, TAIL)],
                        out.at[cid].at[pl.ds(NS * BR, TAIL)])


def _spmm_kernel():
    NBUF = 2
    PAIRS = (NCH - 1) // NBUF

    def body(h_hbm, pck_hbm, zrows_hbm, agg_out,
             pck, sb0, sb1, db0, db1, idx64, idx16, r0, r1,
             g0, g1, s0, s1, agg_sh):
        cid = lax.axis_index("c")
        sid = lax.axis_index("s")
        rows = (r0, r1)
        srcb = (sb0, sb1)
        dstb = (db0, db1)
        gsem = (g0, g1)
        ssem = (s0, s1)
        wid = sid * NC + cid

        pltpu.sync_copy(zrows_hbm, r0)
        _init_shared(sid, r0, sb0, idx64, idx16, agg_sh)
        pltpu.sync_copy(pck_hbm.at[wid], pck)
        plsc.subcore_barrier()

        def unpack(i, k):
            for t in range(CH // 16):
                v = pck[i, pl.ds(16 * t, 16)]
                dstb[k][pl.ds(16 * t, 16)] = lax.bitwise_and(v, 16383)
                srcb[k][pl.ds(16 * t, 16)] = lax.shift_right_logical(v, 14)

        for k in range(NBUF):
            unpack(k, k)
            pltpu.async_copy(h_hbm.at[srcb[k]], rows[k], gsem[k])

        def step(m, carry):
            for k in range(NBUF):
                pltpu.make_async_copy(h_hbm.at[srcb[k]], rows[k],
                                      gsem[k]).wait()
                pltpu.async_copy(rows[k], agg_sh.at[dstb[k]], ssem[k],
                                 add=True)
            for k in range(NBUF):
                pltpu.make_async_copy(rows[k], agg_sh.at[dstb[k]],
                                      ssem[k]).wait()
                nxt = jnp.minimum(NBUF * m + NBUF + k, NCH - 1)
                unpack(nxt, k)
                pltpu.async_copy(h_hbm.at[srcb[k]], rows[k], gsem[k])
            return carry

        lax.fori_loop(0, PAIRS, step, 0)
        pltpu.make_async_copy(h_hbm.at[srcb[0]], rows[0], gsem[0]).wait()
        pltpu.async_copy(rows[0], agg_sh.at[dstb[0]], ssem[0], add=True)
        pltpu.make_async_copy(rows[0], agg_sh.at[dstb[0]], ssem[0]).wait()
        pltpu.make_async_copy(h_hbm.at[srcb[1]], rows[1], gsem[1]).wait()

        plsc.subcore_barrier()
        _write_shared(cid, sid, r0, sb0, idx64, idx16, agg_sh, agg_out)

    return pl.kernel(
        body,
        out_type=(jax.ShapeDtypeStruct((NC, N, D), jnp.float32),),
        mesh=_sc_mesh(),
        scratch_types=(
            pltpu.VMEM((NCH, CH), jnp.int32),
            pltpu.VMEM((CH,), jnp.int32),
            pltpu.VMEM((CH,), jnp.int32),
            pltpu.VMEM((CH,), jnp.int32),
            pltpu.VMEM((CH,), jnp.int32),
            pltpu.VMEM((64,), jnp.int32),
            pltpu.VMEM((TAIL,), jnp.int32),
            pltpu.VMEM((CH, D), jnp.float32),
            pltpu.VMEM((CH, D), jnp.float32),
            pltpu.SemaphoreType.DMA,
            pltpu.SemaphoreType.DMA,
            pltpu.SemaphoreType.DMA,
            pltpu.SemaphoreType.DMA,
            pltpu.VMEM_SHARED((N, D), jnp.float32),
        ))


def _deg_kernel():
    def body(dst_hbm, zrows_hbm, ones_hbm, deg_out,
             dstv, idx80, idx64, idx16, rowsz, onesb, ssem, deg_sh):
        cid = lax.axis_index("c")
        sid = lax.axis_index("s")
        wid = sid * NC + cid

        pltpu.sync_copy(zrows_hbm, rowsz)
        pltpu.sync_copy(ones_hbm, onesb)
        _init_shared(sid, rowsz, idx80, idx64, idx16, deg_sh)
        pltpu.sync_copy(dst_hbm.at[wid], dstv)
        plsc.subcore_barrier()

        K = 5

        def step(m, carry):
            for k in range(K):
                i = K * m + k
                pltpu.async_copy(onesb, deg_sh.at[dstv.at[i]], ssem,
                                 add=True)
            for k in range(K):
                i = K * m + k
                pltpu.make_async_copy(onesb, deg_sh.at[dstv.at[i]],
                                      ssem).wait()
            return carry

        lax.fori_loop(0, NCH // K, step, 0)

        plsc.subcore_barrier()
        _write_shared(cid, sid, rowsz, idx80, idx64, idx16, deg_sh, deg_out)

    return pl.kernel(
        body,
        out_type=(jax.ShapeDtypeStruct((NC, N, D), jnp.float32),),
        mesh=_sc_mesh(),
        scratch_types=(
            pltpu.VMEM((NCH, CH), jnp.int32),
            pltpu.VMEM((CH,), jnp.int32),
            pltpu.VMEM((64,), jnp.int32),
            pltpu.VMEM((TAIL,), jnp.int32),
            pltpu.VMEM((CH, D), jnp.float32),
            pltpu.VMEM((CH, D), jnp.float32),
            pltpu.SemaphoreType.DMA,
            pltpu.VMEM_SHARED((N, D), jnp.float32),
        ))


@functools.cache
def _get_spmm():
    return _spmm_kernel()


@functools.cache
def _get_deg():
    return _deg_kernel()


def _spmm_call(h, pck, zrows):
    return _get_spmm()(h, pck, zrows)[0]


def _deg_call(dst, zrows, ones):
    return _get_deg()(dst, zrows, ones)[0]


def _dense_body(h_ref, agg_ref, deg_ref, ws_ref, wn_ref, b_ref, out_ref, *, relu):
    agg = agg_ref[0] + agg_ref[1]
    deg = jnp.maximum(deg_ref[0, :, 0:1] + deg_ref[1, :, 0:1], 1.0)
    y = (jnp.dot(h_ref[...], ws_ref[...], preferred_element_type=jnp.float32)
         + jnp.dot(agg / deg, wn_ref[...], preferred_element_type=jnp.float32)
         + b_ref[...])
    out_ref[...] = jnp.maximum(y, 0.0) if relu else y


def _dense(h, aggp, degp, ws, wn, bias, relu):
    return pl.pallas_call(
        functools.partial(_dense_body, relu=relu),
        grid=(N // RB,),
        in_specs=[
            pl.BlockSpec((RB, D), lambda i: (i, 0)),
            pl.BlockSpec((NC, RB, D), lambda i: (0, i, 0)),
            pl.BlockSpec((NC, RB, D), lambda i: (0, i, 0)),
            pl.BlockSpec((D, D), lambda i: (0, 0)),
            pl.BlockSpec((D, D), lambda i: (0, 0)),
            pl.BlockSpec((1, D), lambda i: (0, 0)),
        ],
        out_specs=pl.BlockSpec((RB, D), lambda i: (i, 0)),
        out_shape=jax.ShapeDtypeStruct((N, D), jnp.float32),
    )(h, aggp, degp, ws, wn, bias)


def _final_body(h_ref, agg_ref, deg_ref, occ_ref, ws_ref, wn_ref, b_ref,
                s_ref, c_ref):
    i = pl.program_id(0)

    @pl.when(i == 0)
    def _():
        s_ref[...] = jnp.zeros_like(s_ref)
        c_ref[...] = jnp.zeros_like(c_ref)

    agg = agg_ref[0] + agg_ref[1]
    deg = jnp.maximum(deg_ref[0, :, 0:1] + deg_ref[1, :, 0:1], 1.0)
    y = (jnp.dot(h_ref[...], ws_ref[...], preferred_element_type=jnp.float32)
         + jnp.dot(agg / deg, wn_ref[...], preferred_element_type=jnp.float32)
         + b_ref[...])
    occ = occ_ref[0, 0, :]
    onehot = (occ[None, :] ==
              lax.broadcasted_iota(jnp.int32, (NSONG, RB), 0)).astype(jnp.float32)
    s_ref[...] += jnp.dot(onehot, y, preferred_element_type=jnp.float32)
    c_ref[...] = c_ref[...] + jnp.sum(onehot, axis=1, keepdims=True)


def _final(h, aggp, degp, occ3, ws, wn, bias):
    return pl.pallas_call(
        _final_body,
        grid=(N // RB,),
        in_specs=[
            pl.BlockSpec((RB, D), lambda i: (i, 0)),
            pl.BlockSpec((NC, RB, D), lambda i: (0, i, 0)),
            pl.BlockSpec((NC, RB, D), lambda i: (0, i, 0)),
            pl.BlockSpec((1, 1, RB), lambda i: (i, 0, 0)),
            pl.BlockSpec((D, D), lambda i: (0, 0)),
            pl.BlockSpec((D, D), lambda i: (0, 0)),
            pl.BlockSpec((1, D), lambda i: (0, 0)),
        ],
        out_specs=[pl.BlockSpec((NSONG, D), lambda i: (0, 0)),
                   pl.BlockSpec((NSONG, D), lambda i: (0, 0))],
        out_shape=[jax.ShapeDtypeStruct((NSONG, D), jnp.float32),
                   jax.ShapeDtypeStruct((NSONG, D), jnp.float32)],
    )(h, aggp, degp, occ3, ws, wn, bias)


def _head_body(s1_ref, c1_ref, s2_ref, c2_ref,
               wp1_ref, bp1_ref, wp2_ref, bp2_ref, out_ref):
    s1 = s1_ref[...] / jnp.maximum(c1_ref[...], 1.0)
    s2 = s2_ref[...] / jnp.maximum(c2_ref[...], 1.0)

    def proj(s):
        t = jnp.maximum(
            jnp.dot(s, wp1_ref[...], preferred_element_type=jnp.float32)
            + bp1_ref[...], 0.0)
        return (jnp.dot(t, wp2_ref[...], preferred_element_type=jnp.float32)
                + bp2_ref[...])

    z = jnp.concatenate([proj(s1), proj(s2)], axis=0)
    z = z / jnp.sqrt(jnp.sum(z * z, axis=1, keepdims=True))
    sim = lax.dot_general(z, z, (((1,), (1,)), ((), ())),
                          preferred_element_type=jnp.float32) / TEMP
    n2 = 2 * NSONG
    row = lax.broadcasted_iota(jnp.int32, (n2, n2), 0)
    col = lax.broadcasted_iota(jnp.int32, (n2, n2), 1)
    sim = jnp.where(row == col, -1000000000.0, sim)
    m = jnp.max(sim, axis=1, keepdims=True)
    logp = sim - (jnp.log(jnp.sum(jnp.exp(sim - m), axis=1, keepdims=True)) + m)
    lbl = jnp.where(row < NSONG, row + NSONG, row - NSONG)
    loss = -jnp.sum(jnp.where(col == lbl, logp, 0.0)) / n2
    out_ref[...] = jnp.reshape(loss, (1, 1))


def _head(s1, c1, s2, c2, wp1, bp1, wp2, bp2):
    full = lambda s: pl.BlockSpec(s, lambda: (0,) * len(s))
    return pl.pallas_call(
        _head_body,
        in_specs=[full((NSONG, D)), full((NSONG, D)),
                  full((NSONG, D)), full((NSONG, D)),
                  full((D, D)), full((1, D)), full((D, PROJ)), full((1, PROJ))],
        out_specs=full((1, 1)),
        out_shape=jax.ShapeDtypeStruct((1, 1), jnp.float32),
    )(s1, c1, s2, c2, wp1, bp1, wp2, bp2)


def kernel(x1, edge_index1, occ_batch1, x2, edge_index2, occ_batch2,
           Wself, Wneigh, b, Wp1, bp1, Wp2, bp2):
    zrows = jnp.zeros((CH, D), jnp.float32)
    ones = jnp.ones((CH, D), jnp.float32)

    def song_emb(x, ei, occ):
        src = ei[0].astype(jnp.int32)
        dst = ei[1].astype(jnp.int32)
        pck = ((src << 14) | dst).reshape(NW, NCH, CH)
        dst3 = dst.reshape(NW, NCH, CH)
        degp = _deg_call(dst3, zrows, ones)
        aggp = _spmm_call(x, pck, zrows)
        h = _dense(x, aggp, degp, Wself[0], Wneigh[0], b[0][None], True)
        aggp2 = _spmm_call(h, pck, zrows)
        h = _dense(h, aggp2, degp, Wself[1], Wneigh[1], b[1][None], True)
        aggp3 = _spmm_call(h, pck, zrows)
        occ3 = occ.astype(jnp.int32).reshape(N // RB, 1, RB)
        return _final(h, aggp3, degp, occ3, Wself[2], Wneigh[2], b[2][None])

    s1, c1 = song_emb(x1, edge_index1, occ_batch1)
    s2, c2 = song_emb(x2, edge_index2, occ_batch2)
    loss = _head(s1, c1, s2, c2, Wp1, bp1[None], Wp2, bp2[None])
    return loss[0, 0]


# speedup vs baseline: 6.5849x; 1.2175x over previous
import functools

import jax
import jax.numpy as jnp
from jax import lax
from jax.experimental import pallas as pl
from jax.experimental.pallas import tpu as pltpu
from jax.experimental.pallas import tpu_sc as plsc

N = 10000
E = 320000
D = 128
PROJ = 64
NSONG = 64
TEMP = 0.5

NC = 2
NS = 16
NW = NC * NS
EPW = E // NW
CH = 80
NCH = EPW // CH
NBUF = 3
BR = 624
TAIL = N - NS * BR

RB = 1000


def _sc_mesh():
    return plsc.VectorSubcoreMesh(core_axis_name="c", subcore_axis_name="s")


def _fill_idx(idxbuf, start, n):
    for k in range(n // 16):
        idxbuf[pl.ds(16 * k, 16)] = (start + 16 * k) + lax.iota(jnp.int32, 16)


_PIECES = [(k * CH, CH) for k in range(BR // CH)] + [(BR - BR % CH, BR % CH)]


def _init_shared(sid, rows, idx80, idx64, idx16, sh):
    row0 = sid * BR
    for (o, n) in _PIECES:
        buf = {CH: idx80, 64: idx64, TAIL: idx16}[n]
        _fill_idx(buf, row0 + o, n)
        pltpu.sync_copy(rows.at[pl.ds(0, n)], sh.at[buf])

    @pl.when(sid == NS - 1)
    def _():
        _fill_idx(idx16, NS * BR, TAIL)
        pltpu.sync_copy(rows.at[pl.ds(0, TAIL)], sh.at[idx16])


def _write_shared(cid, sid, rows, idx80, idx64, idx16, sh, out):
    row0 = pl.multiple_of(sid * BR, 8)
    for (o, n) in _PIECES:
        buf = {CH: idx80, 64: idx64, TAIL: idx16}[n]
        _fill_idx(buf, row0 + o, n)
        pltpu.sync_copy(sh.at[buf], rows.at[pl.ds(0, n)])
        pltpu.sync_copy(rows.at[pl.ds(0, n)], out.at[cid].at[pl.ds(row0 + o, n)])

    @pl.when(sid == NS - 1)
    def _():
        _fill_idx(idx16, NS * BR, TAIL)
        pltpu.sync_copy(sh.at[idx16], rows.at[pl.ds(0, TAIL)])
        pltpu.sync_copy(rows.at[pl.ds(0, TAIL)],
                        out.at[cid].at[pl.ds(NS * BR, TAIL)])


def _spmm_kernel():
    STEPS = (NCH - 2) // NBUF

    def body(h_hbm, pck_hbm, zrows_hbm, agg_out,
             sb0, sb1, sb2, db0, db1, db2, ib0, ib1, ib2,
             idx80, idx64, idx16, r0, r1, r2,
             g0, g1, g2, s0, s1, s2, i0, i1, i2, agg_sh):
        cid = lax.axis_index("c")
        sid = lax.axis_index("s")
        rows = (r0, r1, r2)
        srcb = (sb0, sb1, sb2)
        dstb = (db0, db1, db2)
        ibuf = (ib0, ib1, ib2)
        gsem = (g0, g1, g2)
        ssem = (s0, s1, s2)
        isem = (i0, i1, i2)
        wid = sid * NC + cid

        pltpu.sync_copy(zrows_hbm, r0)
        _init_shared(sid, r0, idx80, idx64, idx16, agg_sh)
        plsc.subcore_barrier()

        def unpack(k):
            for t in range(CH // 16):
                v = ibuf[k][pl.ds(16 * t, 16)]
                dstb[k][pl.ds(16 * t, 16)] = lax.bitwise_and(v, 16383)
                srcb[k][pl.ds(16 * t, 16)] = lax.shift_right_logical(v, 14)

        for k in range(NBUF):
            pltpu.sync_copy(pck_hbm.at[wid].at[k], ibuf[k])
            unpack(k)
            pltpu.async_copy(h_hbm.at[srcb[k]], rows[k], gsem[k])

        def step(m, carry):
            for k in range(NBUF):
                nxt = jnp.minimum(NBUF * m + NBUF + k, NCH - 1)
                pltpu.async_copy(pck_hbm.at[wid].at[nxt], ibuf[k], isem[k])
                pltpu.make_async_copy(h_hbm.at[srcb[k]], rows[k],
                                      gsem[k]).wait()
                pltpu.async_copy(rows[k], agg_sh.at[dstb[k]], ssem[k],
                                 add=True)
            for k in range(NBUF):
                nxt = jnp.minimum(NBUF * m + NBUF + k, NCH - 1)
                pltpu.make_async_copy(rows[k], agg_sh.at[dstb[k]],
                                      ssem[k]).wait()
                pltpu.make_async_copy(pck_hbm.at[wid].at[nxt], ibuf[k],
                                      isem[k]).wait()
                unpack(k)
                pltpu.async_copy(h_hbm.at[srcb[k]], rows[k], gsem[k])
            return carry

        lax.fori_loop(0, STEPS, step, 0)
        for k in range(2):
            pltpu.make_async_copy(h_hbm.at[srcb[k]], rows[k], gsem[k]).wait()
            pltpu.async_copy(rows[k], agg_sh.at[dstb[k]], ssem[k], add=True)
        pltpu.make_async_copy(h_hbm.at[srcb[2]], rows[2], gsem[2]).wait()
        for k in range(2):
            pltpu.make_async_copy(rows[k], agg_sh.at[dstb[k]], ssem[k]).wait()

        plsc.subcore_barrier()
        _write_shared(cid, sid, r0, idx80, idx64, idx16, agg_sh, agg_out)

    return pl.kernel(
        body,
        out_type=(jax.ShapeDtypeStruct((NC, N, D), jnp.float32),),
        mesh=_sc_mesh(),
        scratch_types=(
            (pltpu.VMEM((CH,), jnp.int32),) * NBUF
            + (pltpu.VMEM((CH,), jnp.int32),) * NBUF
            + (pltpu.VMEM((CH,), jnp.int32),) * NBUF
            + (pltpu.VMEM((CH,), jnp.int32),
               pltpu.VMEM((64,), jnp.int32),
               pltpu.VMEM((TAIL,), jnp.int32))
            + (pltpu.VMEM((CH, D), jnp.float32),) * NBUF
            + (pltpu.SemaphoreType.DMA,) * (3 * NBUF)
            + (pltpu.VMEM_SHARED((N, D), jnp.float32),)
        ))


def _deg_kernel():
    def body(dst_hbm, zrows_hbm, ones_hbm, deg_out,
             dstv, idx80, idx64, idx16, rowsz, onesb, ssem, deg_sh):
        cid = lax.axis_index("c")
        sid = lax.axis_index("s")
        wid = sid * NC + cid

        pltpu.sync_copy(zrows_hbm, rowsz)
        pltpu.sync_copy(ones_hbm, onesb)
        _init_shared(sid, rowsz, idx80, idx64, idx16, deg_sh)
        pltpu.sync_copy(dst_hbm.at[wid], dstv)
        plsc.subcore_barrier()

        K = 5

        def step(m, carry):
            for k in range(K):
                i = K * m + k
                pltpu.async_copy(onesb, deg_sh.at[dstv.at[i]], ssem,
                                 add=True)
            for k in range(K):
                i = K * m + k
                pltpu.make_async_copy(onesb, deg_sh.at[dstv.at[i]],
                                      ssem).wait()
            return carry

        lax.fori_loop(0, NCH // K, step, 0)

        plsc.subcore_barrier()
        _write_shared(cid, sid, rowsz, idx80, idx64, idx16, deg_sh, deg_out)

    return pl.kernel(
        body,
        out_type=(jax.ShapeDtypeStruct((NC, N, D), jnp.float32),),
        mesh=_sc_mesh(),
        scratch_types=(
            pltpu.VMEM((NCH, CH), jnp.int32),
            pltpu.VMEM((CH,), jnp.int32),
            pltpu.VMEM((64,), jnp.int32),
            pltpu.VMEM((TAIL,), jnp.int32),
            pltpu.VMEM((CH, D), jnp.float32),
            pltpu.VMEM((CH, D), jnp.float32),
            pltpu.SemaphoreType.DMA,
            pltpu.VMEM_SHARED((N, D), jnp.float32),
        ))


@functools.cache
def _get_spmm():
    return _spmm_kernel()


@functools.cache
def _get_deg():
    return _deg_kernel()


def _spmm_call(h, pck, zrows):
    return _get_spmm()(h, pck, zrows)[0]


def _deg_call(dst, zrows, ones):
    return _get_deg()(dst, zrows, ones)[0]


def _dense_body(h_ref, agg_ref, deg_ref, ws_ref, wn_ref, b_ref, out_ref, *, relu):
    agg = agg_ref[0] + agg_ref[1]
    deg = jnp.maximum(deg_ref[0, :, 0:1] + deg_ref[1, :, 0:1], 1.0)
    y = (jnp.dot(h_ref[...], ws_ref[...], preferred_element_type=jnp.float32)
         + jnp.dot(agg / deg, wn_ref[...], preferred_element_type=jnp.float32)
         + b_ref[...])
    out_ref[...] = jnp.maximum(y, 0.0) if relu else y


def _dense(h, aggp, degp, ws, wn, bias, relu):
    return pl.pallas_call(
        functools.partial(_dense_body, relu=relu),
        grid=(N // RB,),
        in_specs=[
            pl.BlockSpec((RB, D), lambda i: (i, 0)),
            pl.BlockSpec((NC, RB, D), lambda i: (0, i, 0)),
            pl.BlockSpec((NC, RB, D), lambda i: (0, i, 0)),
            pl.BlockSpec((D, D), lambda i: (0, 0)),
            pl.BlockSpec((D, D), lambda i: (0, 0)),
            pl.BlockSpec((1, D), lambda i: (0, 0)),
        ],
        out_specs=pl.BlockSpec((RB, D), lambda i: (i, 0)),
        out_shape=jax.ShapeDtypeStruct((N, D), jnp.float32),
    )(h, aggp, degp, ws, wn, bias)


def _final_body(h_ref, agg_ref, deg_ref, occ_ref, ws_ref, wn_ref, b_ref,
                s_ref, c_ref):
    i = pl.program_id(0)

    @pl.when(i == 0)
    def _():
        s_ref[...] = jnp.zeros_like(s_ref)
        c_ref[...] = jnp.zeros_like(c_ref)

    agg = agg_ref[0] + agg_ref[1]
    deg = jnp.maximum(deg_ref[0, :, 0:1] + deg_ref[1, :, 0:1], 1.0)
    y = (jnp.dot(h_ref[...], ws_ref[...], preferred_element_type=jnp.float32)
         + jnp.dot(agg / deg, wn_ref[...], preferred_element_type=jnp.float32)
         + b_ref[...])
    occ = occ_ref[0, 0, :]
    onehot = (occ[None, :] ==
              lax.broadcasted_iota(jnp.int32, (NSONG, RB), 0)).astype(jnp.float32)
    s_ref[...] += jnp.dot(onehot, y, preferred_element_type=jnp.float32)
    c_ref[...] = c_ref[...] + jnp.sum(onehot, axis=1, keepdims=True)


def _final(h, aggp, degp, occ3, ws, wn, bias):
    return pl.pallas_call(
        _final_body,
        grid=(N // RB,),
        in_specs=[
            pl.BlockSpec((RB, D), lambda i: (i, 0)),
            pl.BlockSpec((NC, RB, D), lambda i: (0, i, 0)),
            pl.BlockSpec((NC, RB, D), lambda i: (0, i, 0)),
            pl.BlockSpec((1, 1, RB), lambda i: (i, 0, 0)),
            pl.BlockSpec((D, D), lambda i: (0, 0)),
            pl.BlockSpec((D, D), lambda i: (0, 0)),
            pl.BlockSpec((1, D), lambda i: (0, 0)),
        ],
        out_specs=[pl.BlockSpec((NSONG, D), lambda i: (0, 0)),
                   pl.BlockSpec((NSONG, D), lambda i: (0, 0))],
        out_shape=[jax.ShapeDtypeStruct((NSONG, D), jnp.float32),
                   jax.ShapeDtypeStruct((NSONG, D), jnp.float32)],
    )(h, aggp, degp, occ3, ws, wn, bias)


def _head_body(s1_ref, c1_ref, s2_ref, c2_ref,
               wp1_ref, bp1_ref, wp2_ref, bp2_ref, out_ref):
    s1 = s1_ref[...] / jnp.maximum(c1_ref[...], 1.0)
    s2 = s2_ref[...] / jnp.maximum(c2_ref[...], 1.0)

    def proj(s):
        t = jnp.maximum(
            jnp.dot(s, wp1_ref[...], preferred_element_type=jnp.float32)
            + bp1_ref[...], 0.0)
        return (jnp.dot(t, wp2_ref[...], preferred_element_type=jnp.float32)
                + bp2_ref[...])

    z = jnp.concatenate([proj(s1), proj(s2)], axis=0)
    z = z / jnp.sqrt(jnp.sum(z * z, axis=1, keepdims=True))
    sim = lax.dot_general(z, z, (((1,), (1,)), ((), ())),
                          preferred_element_type=jnp.float32) / TEMP
    n2 = 2 * NSONG
    row = lax.broadcasted_iota(jnp.int32, (n2, n2), 0)
    col = lax.broadcasted_iota(jnp.int32, (n2, n2), 1)
    sim = jnp.where(row == col, -1000000000.0, sim)
    m = jnp.max(sim, axis=1, keepdims=True)
    logp = sim - (jnp.log(jnp.sum(jnp.exp(sim - m), axis=1, keepdims=True)) + m)
    lbl = jnp.where(row < NSONG, row + NSONG, row - NSONG)
    loss = -jnp.sum(jnp.where(col == lbl, logp, 0.0)) / n2
    out_ref[...] = jnp.reshape(loss, (1, 1))


def _head(s1, c1, s2, c2, wp1, bp1, wp2, bp2):
    full = lambda s: pl.BlockSpec(s, lambda: (0,) * len(s))
    return pl.pallas_call(
        _head_body,
        in_specs=[full((NSONG, D)), full((NSONG, D)),
                  full((NSONG, D)), full((NSONG, D)),
                  full((D, D)), full((1, D)), full((D, PROJ)), full((1, PROJ))],
        out_specs=full((1, 1)),
        out_shape=jax.ShapeDtypeStruct((1, 1), jnp.float32),
    )(s1, c1, s2, c2, wp1, bp1, wp2, bp2)


def kernel(x1, edge_index1, occ_batch1, x2, edge_index2, occ_batch2,
           Wself, Wneigh, b, Wp1, bp1, Wp2, bp2):
    zrows = jnp.zeros((CH, D), jnp.float32)
    ones = jnp.ones((CH, D), jnp.float32)

    def song_emb(x, ei, occ):
        src = ei[0].astype(jnp.int32)
        dst = ei[1].astype(jnp.int32)
        pck = ((src << 14) | dst).reshape(NW, NCH, CH)
        dst3 = dst.reshape(NW, NCH, CH)
        degp = _deg_call(dst3, zrows, ones)
        aggp = _spmm_call(x, pck, zrows)
        h = _dense(x, aggp, degp, Wself[0], Wneigh[0], b[0][None], True)
        aggp2 = _spmm_call(h, pck, zrows)
        h = _dense(h, aggp2, degp, Wself[1], Wneigh[1], b[1][None], True)
        aggp3 = _spmm_call(h, pck, zrows)
        occ3 = occ.astype(jnp.int32).reshape(N // RB, 1, RB)
        return _final(h, aggp3, degp, occ3, Wself[2], Wneigh[2], b[2][None])

    s1, c1 = song_emb(x1, edge_index1, occ_batch1)
    s2, c2 = song_emb(x2, edge_index2, occ_batch2)
    loss = _head(s1, c1, s2, c2, Wp1, bp1[None], Wp2, bp2[None])
    return loss[0, 0]
